# baseline (speedup 1.0000x reference)
import functools

import jax
import jax.numpy as jnp
from jax import lax
from jax.experimental import pallas as pl
from jax.experimental.pallas import tpu as pltpu, tpu_sc as plsc

B = 16384
N_DENSE = 13
N_CAT = 26
VOCAB = 100000
VROWS = VOCAB + 2
D = 32
LINE = 128
NLINES = N_CAT * VROWS * D // LINE
EMB_COLS = N_CAT * D
OUT_COLS = N_DENSE + EMB_COLS

NC = 2
NS = 16
NW = NC * NS
BPW = B // NW
C = 16
NCH = BPW // C
HALF = N_CAT // 2

RBLK = 512


def _bn_body(x_ref, g_ref, b_ref, o_ref):
    x = x_ref[...]
    mean = jnp.mean(x, axis=0, keepdims=True)
    cen = x - mean
    var = jnp.mean(cen * cen, axis=0, keepdims=True)
    o_ref[...] = cen * lax.rsqrt(var + 1e-5) * g_ref[...] + b_ref[...]


def _batchnorm(x, gamma, beta):
    return pl.pallas_call(
        _bn_body,
        out_shape=jax.ShapeDtypeStruct((B, N_DENSE), jnp.float32),
    )(x, gamma.reshape(1, N_DENSE), beta.reshape(1, N_DENSE))


def _concat_body(x_ref, e_ref, o_ref):
    o_ref[...] = jnp.concatenate([x_ref[...], e_ref[...]], axis=1)


def _concat(xres, emb):
    return pl.pallas_call(
        _concat_body,
        grid=(B // RBLK,),
        in_specs=[
            pl.BlockSpec((RBLK, N_DENSE), lambda i: (i, 0)),
            pl.BlockSpec((RBLK, EMB_COLS), lambda i: (i, 0)),
        ],
        out_specs=pl.BlockSpec((RBLK, OUT_COLS), lambda i: (i, 0)),
        out_shape=jax.ShapeDtypeStruct((B, OUT_COLS), jnp.float32),
    )(xres, emb)


def _sc_body(tab, cat, emb, catc, qbuf, gpad, ebuf, sem0, sem1):
    c = lax.axis_index("c")
    s = lax.axis_index("s")
    wid = s * NC + c
    base = wid * BPW
    sems = (sem0, sem1)

    def fire(h):
        for ii in range(HALF):
            i = h * HALF + ii
            qbuf[i, :] = (catc[i, :] + i * VROWS) >> 2
        handles = []
        for ii in range(HALF):
            i = h * HALF + ii
            handles.append(
                pltpu.async_copy(tab.at[qbuf.at[i]], gpad.at[h, ii], sems[h])
            )
        return handles

    def extract(h):
        for ii in range(HALF):
            i = h * HALF + ii
            col0 = D * i
            iv = catc[i, :]
            for r in range(C):
                sub = ((iv[r] + i * VROWS) & 3) * D
                ebuf[r, pl.ds(col0, 16)] = gpad[h, ii, r, pl.ds(sub, 16)]
                ebuf[r, pl.ds(col0 + 16, 16)] = gpad[h, ii, r, pl.ds(sub + 16, 16)]

    def chunk(ch, _):
        b0 = base + ch * C
        pltpu.sync_copy(cat.at[(b0 // C)], catc)
        ha = fire(0)
        hb = fire(1)
        for hnd in ha:
            hnd.wait()
        extract(0)
        for hnd in hb:
            hnd.wait()
        extract(1)
        pltpu.sync_copy(ebuf, emb.at[pl.ds(b0, C), :])
        return 0

    lax.fori_loop(0, NCH, chunk, 0)


@functools.cache
def _sc_call():
    return functools.partial(
        pl.kernel,
        out_type=jax.ShapeDtypeStruct((B, EMB_COLS), jnp.float32),
        mesh=plsc.VectorSubcoreMesh(core_axis_name="c", subcore_axis_name="s"),
        scratch_types=[
            pltpu.VMEM((N_CAT, C), jnp.int32),
            pltpu.VMEM((N_CAT, C), jnp.int32),
            pltpu.VMEM((2, HALF, C, LINE), jnp.float32),
            pltpu.VMEM((C, EMB_COLS), jnp.float32),
            pltpu.SemaphoreType.DMA,
            pltpu.SemaphoreType.DMA,
        ],
    )(_sc_body)


def kernel(x, categorical, tables, gamma, beta):
    cat4 = (
        categorical.astype(jnp.int32)
        .reshape(B // C, C, N_CAT)
        .transpose(0, 2, 1)
    )
    tab128 = jnp.pad(tables, ((0, 0), (0, VROWS - VOCAB - 1), (0, 0))).reshape(
        NLINES, LINE
    )
    emb = _sc_call()(tab128, cat4)
    xres = _batchnorm(x, gamma, beta)
    return _concat(xres, emb)


# speedup vs baseline: 7.3185x; 7.3185x over previous
import functools

import jax
import jax.numpy as jnp
from jax import lax
from jax.experimental import pallas as pl
from jax.experimental.pallas import tpu as pltpu, tpu_sc as plsc

B = 16384
N_DENSE = 13
N_CAT = 26
VOCAB = 100000
D = 32
LINE = 128
VCHUNK = 4096
NCHK = 25
LPC = VCHUNK // 4
CPF = NCHK * LPC
NLINES = N_CAT * CPF
EMB_COLS = N_CAT * D
OUT_COLS = N_DENSE + EMB_COLS

NC = 2
NS = 16
NW = NC * NS
BPW = B // NW
C = 16
NCH = BPW // C
HALF = N_CAT // 2

RBLK = 512


def _repack_body(x_ref, o_ref):
    x = x_ref[...]
    o_ref[...] = jnp.concatenate(
        [x[:, LPC * r : LPC * (r + 1)].T for r in range(4)], axis=1
    )


def _repack(tab_t):
    return pl.pallas_call(
        _repack_body,
        grid=(N_CAT, NCHK),
        in_specs=[pl.BlockSpec((D, VCHUNK), lambda i, j: (i, j))],
        out_specs=pl.BlockSpec((LPC, LINE), lambda i, j: (i * NCHK + j, 0)),
        out_shape=jax.ShapeDtypeStruct((NLINES, LINE), jnp.float32),
    )(tab_t)


def _bn_body(x_ref, g_ref, b_ref, o_ref):
    x = x_ref[...]
    mean = jnp.mean(x, axis=0, keepdims=True)
    cen = x - mean
    var = jnp.mean(cen * cen, axis=0, keepdims=True)
    o_ref[...] = cen * lax.rsqrt(var + 1e-5) * g_ref[...] + b_ref[...]


def _batchnorm(x, gamma, beta):
    return pl.pallas_call(
        _bn_body,
        out_shape=jax.ShapeDtypeStruct((B, N_DENSE), jnp.float32),
    )(x, gamma.reshape(1, N_DENSE), beta.reshape(1, N_DENSE))


def _concat_body(x_ref, e_ref, o_ref):
    o_ref[...] = jnp.concatenate([x_ref[...], e_ref[...]], axis=1)


def _concat(xres, emb):
    return pl.pallas_call(
        _concat_body,
        grid=(B // RBLK,),
        in_specs=[
            pl.BlockSpec((RBLK, N_DENSE), lambda i: (i, 0)),
            pl.BlockSpec((RBLK, EMB_COLS), lambda i: (i, 0)),
        ],
        out_specs=pl.BlockSpec((RBLK, OUT_COLS), lambda i: (i, 0)),
        out_shape=jax.ShapeDtypeStruct((B, OUT_COLS), jnp.float32),
    )(xres, emb)


def _sc_body(tab, cat, emb, catc, qbuf, gpad, ebuf, sem0, sem1):
    c = lax.axis_index("c")
    s = lax.axis_index("s")
    wid = s * NC + c
    base = wid * BPW
    sems = (sem0, sem1)

    def fire(h):
        for ii in range(HALF):
            i = h * HALF + ii
            v = catc[i, :]
            qbuf[i, :] = ((v >> 12) << 10) + (v & (LPC - 1)) + i * CPF
        handles = []
        for ii in range(HALF):
            i = h * HALF + ii
            handles.append(
                pltpu.async_copy(tab.at[qbuf.at[i]], gpad.at[h, ii], sems[h])
            )
        return handles

    def extract(h):
        for ii in range(HALF):
            i = h * HALF + ii
            col0 = D * i
            iv = catc[i, :]
            for r in range(C):
                sub = ((iv[r] >> 10) & 3) * D
                ebuf[r, pl.ds(col0, 16)] = gpad[h, ii, r, pl.ds(sub, 16)]
                ebuf[r, pl.ds(col0 + 16, 16)] = gpad[h, ii, r, pl.ds(sub + 16, 16)]

    def chunk(ch, _):
        b0 = base + ch * C
        pltpu.sync_copy(cat.at[(b0 // C)], catc)
        ha = fire(0)
        hb = fire(1)
        for hnd in ha:
            hnd.wait()
        extract(0)
        for hnd in hb:
            hnd.wait()
        extract(1)
        pltpu.sync_copy(ebuf, emb.at[pl.ds(b0, C), :])
        return 0

    lax.fori_loop(0, NCH, chunk, 0)


@functools.cache
def _sc_call():
    return functools.partial(
        pl.kernel,
        out_type=jax.ShapeDtypeStruct((B, EMB_COLS), jnp.float32),
        mesh=plsc.VectorSubcoreMesh(core_axis_name="c", subcore_axis_name="s"),
        scratch_types=[
            pltpu.VMEM((N_CAT, C), jnp.int32),
            pltpu.VMEM((N_CAT, C), jnp.int32),
            pltpu.VMEM((2, HALF, C, LINE), jnp.float32),
            pltpu.VMEM((C, EMB_COLS), jnp.float32),
            pltpu.SemaphoreType.DMA,
            pltpu.SemaphoreType.DMA,
        ],
    )(_sc_body)


def kernel(x, categorical, tables, gamma, beta):
    cat4 = (
        categorical.astype(jnp.int32)
        .reshape(B // C, C, N_CAT)
        .transpose(0, 2, 1)
    )
    tab_t = tables.transpose(0, 2, 1).reshape(N_CAT * D, VOCAB + 1)
    tab128 = _repack(tab_t)
    emb = _sc_call()(tab128, cat4)
    xres = _batchnorm(x, gamma, beta)
    return _concat(xres, emb)


# speedup vs baseline: 8.1352x; 1.1116x over previous
import functools

import jax
import jax.numpy as jnp
from jax import lax
from jax.experimental import pallas as pl
from jax.experimental.pallas import tpu as pltpu, tpu_sc as plsc

B = 16384
N_DENSE = 13
N_CAT = 26
VOCAB = 100000
D = 32
LINE = 128
VCHUNK = 4096
NCHK = 25
LPC = VCHUNK // 4
CPF = NCHK * LPC
NLINES = N_CAT * CPF
EMB_COLS = N_CAT * D
OUT_COLS = N_DENSE + EMB_COLS

NC = 2
NS = 16
NW = NC * NS
BPW = B // NW
C = 16
NCH = BPW // C
HALF = N_CAT // 2

RBLK = 512


def _repack_body(x_ref, o_ref):
    x = x_ref[...]
    o_ref[...] = jnp.concatenate(
        [x[:, LPC * r : LPC * (r + 1)].T for r in range(4)], axis=1
    )


def _repack(tab_t, field_base):
    return pl.pallas_call(
        _repack_body,
        grid=(HALF, NCHK),
        in_specs=[pl.BlockSpec((D, VCHUNK), lambda i, j: (i + field_base, j))],
        out_specs=pl.BlockSpec((LPC, LINE), lambda i, j: (i * NCHK + j, 0)),
        out_shape=jax.ShapeDtypeStruct((HALF * CPF, LINE), jnp.float32),
    )(tab_t)


def _bn_body(x_ref, g_ref, b_ref, o_ref):
    x = x_ref[...]
    mean = jnp.mean(x, axis=0, keepdims=True)
    cen = x - mean
    var = jnp.mean(cen * cen, axis=0, keepdims=True)
    o_ref[...] = cen * lax.rsqrt(var + 1e-5) * g_ref[...] + b_ref[...]


def _batchnorm(x, gamma, beta):
    return pl.pallas_call(
        _bn_body,
        out_shape=jax.ShapeDtypeStruct((B, N_DENSE), jnp.float32),
    )(x, gamma.reshape(1, N_DENSE), beta.reshape(1, N_DENSE))


def _concat_body(x_ref, ea_ref, eb_ref, o_ref):
    o_ref[...] = jnp.concatenate([x_ref[...], ea_ref[...], eb_ref[...]], axis=1)


def _concat(xres, emb_a, emb_b):
    return pl.pallas_call(
        _concat_body,
        grid=(B // RBLK,),
        in_specs=[
            pl.BlockSpec((RBLK, N_DENSE), lambda i: (i, 0)),
            pl.BlockSpec((RBLK, EMB_COLS // 2), lambda i: (i, 0)),
            pl.BlockSpec((RBLK, EMB_COLS // 2), lambda i: (i, 0)),
        ],
        out_specs=pl.BlockSpec((RBLK, OUT_COLS), lambda i: (i, 0)),
        out_shape=jax.ShapeDtypeStruct((B, OUT_COLS), jnp.float32),
    )(xres, emb_a, emb_b)


GA = 7


def _sc_body(tab, cat, emb, catc, qbuf, gpad_a, gpad_b, ebuf, sem0, sem1):
    c = lax.axis_index("c")
    s = lax.axis_index("s")
    wid = s * NC + c
    base = wid * BPW

    def fire(fields, gpad, sem):
        for k, i in enumerate(fields):
            v = catc[i, :]
            qbuf[i, :] = ((v >> 12) << 10) + (v & (LPC - 1)) + i * CPF
        handles = []
        for k, i in enumerate(fields):
            handles.append(
                pltpu.async_copy(tab.at[qbuf.at[i]], gpad.at[k], sem)
            )
        return handles

    def extract(fields, gpad):
        for k, i in enumerate(fields):
            col0 = D * i
            iv = catc[i, :]
            for r in range(C):
                sub = ((iv[r] >> 10) & 3) * D
                ebuf[r, pl.ds(col0, 16)] = gpad[k, r, pl.ds(sub, 16)]
                ebuf[r, pl.ds(col0 + 16, 16)] = gpad[k, r, pl.ds(sub + 16, 16)]

    fa = tuple(range(GA))
    fb = tuple(range(GA, HALF))

    def chunk(ch, _):
        b0 = base + ch * C
        pltpu.sync_copy(cat.at[(b0 // C)], catc)
        ha = fire(fa, gpad_a, sem0)
        hb = fire(fb, gpad_b, sem1)
        for hnd in ha:
            hnd.wait()
        extract(fa, gpad_a)
        for hnd in hb:
            hnd.wait()
        extract(fb, gpad_b)
        pltpu.sync_copy(ebuf, emb.at[pl.ds(b0, C), :])
        return 0

    lax.fori_loop(0, NCH, chunk, 0)


@functools.cache
def _sc_call():
    return functools.partial(
        pl.kernel,
        out_type=jax.ShapeDtypeStruct((B, EMB_COLS // 2), jnp.float32),
        mesh=plsc.VectorSubcoreMesh(core_axis_name="c", subcore_axis_name="s"),
        scratch_types=[
            pltpu.VMEM((HALF, C), jnp.int32),
            pltpu.VMEM((HALF, C), jnp.int32),
            pltpu.VMEM((GA, C, LINE), jnp.float32),
            pltpu.VMEM((HALF - GA, C, LINE), jnp.float32),
            pltpu.VMEM((C, EMB_COLS // 2), jnp.float32),
            pltpu.SemaphoreType.DMA,
            pltpu.SemaphoreType.DMA,
        ],
    )(_sc_body)


def _cat_half(categorical, lo):
    return (
        lax.slice_in_dim(categorical.astype(jnp.int32), lo, lo + HALF, axis=1)
        .reshape(B // C, C, HALF)
        .transpose(0, 2, 1)
    )


def kernel(x, categorical, tables, gamma, beta):
    tab_t = tables.transpose(0, 2, 1).reshape(N_CAT * D, VOCAB + 1)
    tab_a = _repack(tab_t, 0)
    emb_a = _sc_call()(tab_a, _cat_half(categorical, 0))
    tab_b = _repack(tab_t, HALF)
    emb_b = _sc_call()(tab_b, _cat_half(categorical, HALF))
    xres = _batchnorm(x, gamma, beta)
    return _concat(xres, emb_a, emb_b)
